# baseline (speedup 1.0000x reference)
import functools

import jax
import jax.numpy as jnp
from jax import lax
from jax.experimental import pallas as pl
from jax.experimental.pallas import tpu as pltpu
from jax.experimental.pallas import tpu_sc as plsc

EMBED = 512
HIDDEN = 1024
STRIPE = 2048

_NUM_WORKERS = 32
_CH = 64
_NB = 2

_CHUNKS = ((1, 1024), (2, 2048), (1, 1024))


def _sc_gather_chunk(table, ids2d, base_stripe, n_stripes):
    mesh = plsc.VectorSubcoreMesh(core_axis_name="c", subcore_axis_name="s")

    @functools.partial(
        pl.kernel,
        out_type=jax.ShapeDtypeStruct((n_stripes * STRIPE, EMBED), jnp.float32),
        mesh=mesh,
        scratch_types=[
            pltpu.VMEM((n_stripes, _CH), jnp.int32),
            pltpu.VMEM((_NB, _CH, EMBED), jnp.float32),
            [pltpu.SemaphoreType.DMA] * _NB,
            [pltpu.SemaphoreType.DMA] * _NB,
        ],
    )
    def k(table_hbm, idx_hbm, out_hbm, idx_v, bufs, gsems, wsems):
        wid = lax.axis_index("s") * 2 + lax.axis_index("c")
        col = wid * _CH
        for c in range(n_stripes):
            pltpu.sync_copy(idx_hbm.at[base_stripe + c, pl.ds(col, _CH)], idx_v.at[c])

        def gather(c, s):
            return pltpu.async_copy(
                table_hbm.at[idx_v.at[c]], bufs.at[s], gsems[s]
            )

        gs = [None] * n_stripes
        ws = [None] * n_stripes
        for s in range(min(_NB, n_stripes)):
            gs[s] = gather(s, s)
        for c in range(n_stripes):
            s = c % _NB
            gs[c].wait()
            ws[c] = pltpu.async_copy(
                bufs.at[s], out_hbm.at[pl.ds(c * STRIPE + col, _CH)], wsems[s]
            )
            if c + _NB < n_stripes:
                ws[c].wait()
                gs[c + _NB] = gather(c + _NB, s)
        for c in range(max(0, n_stripes - _NB), n_stripes):
            ws[c].wait()

    return k(table, ids2d)


def _tc_project_chunk(x, positions, w, b, buf, base_row, n_total, tb):
    rows = x.shape[0]
    base_tile = base_row // tb

    def body(buf_ref, x_ref, pos_ref, w_ref, b_ref, o_ref):
        del buf_ref
        xx = (x_ref[...] + pos_ref[1:2, :]).astype(jnp.bfloat16)
        w16 = w_ref[...].astype(jnp.bfloat16)
        o_ref[...] = (
            jnp.dot(xx, w16, preferred_element_type=jnp.float32) + b_ref[...]
        )

    if buf is None:
        buf = jnp.zeros((8, HIDDEN), jnp.float32)
        aliases = {}
    else:
        aliases = {0: 0}

    return pl.pallas_call(
        body,
        grid=(rows // tb,),
        in_specs=[
            pl.BlockSpec(memory_space=pl.ANY),
            pl.BlockSpec((tb, EMBED), lambda i: (i, 0)),
            pl.BlockSpec((8, EMBED), lambda i: (0, 0)),
            pl.BlockSpec((EMBED, HIDDEN), lambda i: (0, 0)),
            pl.BlockSpec((1, HIDDEN), lambda i: (0, 0)),
        ],
        out_specs=pl.BlockSpec(
            (tb, HIDDEN), lambda i, bt=base_tile: (bt + i, 0)
        ),
        out_shape=jax.ShapeDtypeStruct((n_total, HIDDEN), jnp.float32),
        input_output_aliases=aliases,
    )(buf, x, positions, w, b)


def kernel(input_ids, embed_tokens_w, embed_positions_w, proj_w, proj_b):
    batch, seq = input_ids.shape
    b_total = batch * seq
    ids = input_ids.astype(jnp.int32)
    b2d = proj_b.reshape(1, HIDDEN)

    gathered = []
    base = 0
    for n_stripes, _ in _CHUNKS:
        gathered.append(_sc_gather_chunk(embed_tokens_w, ids, base, n_stripes))
        base += n_stripes

    buf = None
    base_row = 0
    for (n_stripes, tb), g in zip(_CHUNKS, gathered):
        buf = _tc_project_chunk(
            g, embed_positions_w, proj_w, b2d, buf, base_row, b_total, tb
        )
        base_row += n_stripes * STRIPE
    return buf.reshape(batch, seq, HIDDEN)


# speedup vs baseline: 1.0625x; 1.0625x over previous
import functools

import jax
import jax.numpy as jnp
from jax import lax
from jax.experimental import pallas as pl
from jax.experimental.pallas import tpu as pltpu
from jax.experimental.pallas import tpu_sc as plsc

EMBED = 512
HIDDEN = 1024
STRIPE = 2048

_NUM_WORKERS = 32
_CH = 64
_NB = 2

_CHUNKS = ((2, 2048), (2, 2048))


def _sc_gather_chunk(table, ids2d, base_stripe, n_stripes):
    mesh = plsc.VectorSubcoreMesh(core_axis_name="c", subcore_axis_name="s")

    @functools.partial(
        pl.kernel,
        out_type=jax.ShapeDtypeStruct((n_stripes * STRIPE, EMBED), jnp.float32),
        mesh=mesh,
        scratch_types=[
            pltpu.VMEM((n_stripes, _CH), jnp.int32),
            pltpu.VMEM((_NB, _CH, EMBED), jnp.float32),
            [pltpu.SemaphoreType.DMA] * _NB,
            [pltpu.SemaphoreType.DMA] * _NB,
        ],
    )
    def k(table_hbm, idx_hbm, out_hbm, idx_v, bufs, gsems, wsems):
        wid = lax.axis_index("s") * 2 + lax.axis_index("c")
        col = wid * _CH
        for c in range(n_stripes):
            pltpu.sync_copy(idx_hbm.at[base_stripe + c, pl.ds(col, _CH)], idx_v.at[c])

        def gather(c, s):
            return pltpu.async_copy(
                table_hbm.at[idx_v.at[c]], bufs.at[s], gsems[s]
            )

        gs = [None] * n_stripes
        ws = [None] * n_stripes
        for s in range(min(_NB, n_stripes)):
            gs[s] = gather(s, s)
        for c in range(n_stripes):
            s = c % _NB
            gs[c].wait()
            ws[c] = pltpu.async_copy(
                bufs.at[s], out_hbm.at[pl.ds(c * STRIPE + col, _CH)], wsems[s]
            )
            if c + _NB < n_stripes:
                ws[c].wait()
                gs[c + _NB] = gather(c + _NB, s)
        for c in range(max(0, n_stripes - _NB), n_stripes):
            ws[c].wait()

    return k(table, ids2d)


def _tc_project_chunk(x, positions, w, b, buf, base_row, n_total, tb):
    rows = x.shape[0]
    base_tile = base_row // tb

    def body(buf_ref, x_ref, pos_ref, w_ref, b_ref, o_ref):
        del buf_ref
        xx = (x_ref[...] + pos_ref[1:2, :]).astype(jnp.bfloat16)
        w16 = w_ref[...].astype(jnp.bfloat16)
        o_ref[...] = (
            jnp.dot(xx, w16, preferred_element_type=jnp.float32) + b_ref[...]
        )

    if buf is None:
        buf = jnp.zeros((8, HIDDEN), jnp.float32)
        aliases = {}
    else:
        aliases = {0: 0}

    return pl.pallas_call(
        body,
        grid=(rows // tb,),
        in_specs=[
            pl.BlockSpec(memory_space=pl.ANY),
            pl.BlockSpec((tb, EMBED), lambda i: (i, 0)),
            pl.BlockSpec((8, EMBED), lambda i: (0, 0)),
            pl.BlockSpec((EMBED, HIDDEN), lambda i: (0, 0)),
            pl.BlockSpec((1, HIDDEN), lambda i: (0, 0)),
        ],
        out_specs=pl.BlockSpec(
            (tb, HIDDEN), lambda i, bt=base_tile: (bt + i, 0)
        ),
        out_shape=jax.ShapeDtypeStruct((n_total, HIDDEN), jnp.float32),
        input_output_aliases=aliases,
    )(buf, x, positions, w, b)


def kernel(input_ids, embed_tokens_w, embed_positions_w, proj_w, proj_b):
    batch, seq = input_ids.shape
    b_total = batch * seq
    ids = input_ids.astype(jnp.int32)
    b2d = proj_b.reshape(1, HIDDEN)

    gathered = []
    base = 0
    for n_stripes, _ in _CHUNKS:
        gathered.append(_sc_gather_chunk(embed_tokens_w, ids, base, n_stripes))
        base += n_stripes

    buf = None
    base_row = 0
    for (n_stripes, tb), g in zip(_CHUNKS, gathered):
        buf = _tc_project_chunk(
            g, embed_positions_w, proj_w, b2d, buf, base_row, b_total, tb
        )
        base_row += n_stripes * STRIPE
    return buf.reshape(batch, seq, HIDDEN)


# speedup vs baseline: 1.0810x; 1.0174x over previous
import functools

import jax
import jax.numpy as jnp
from jax import lax
from jax.experimental import pallas as pl
from jax.experimental.pallas import tpu as pltpu
from jax.experimental.pallas import tpu_sc as plsc

EMBED = 512
HIDDEN = 1024

_NUM_WORKERS = 32
_N_CHUNKS = 2
_CH = 64
_NB = 2
_TB = 2048


def _sc_gather_chunk(table, ids2d, chunk, rows_per_chunk):
    batch, seq = ids2d.shape
    b_per_w = rows_per_chunk // _NUM_WORKERS
    n_ch = b_per_w // _CH
    rows_per_batch = seq // b_per_w
    mesh = plsc.VectorSubcoreMesh(core_axis_name="c", subcore_axis_name="s")

    @functools.partial(
        pl.kernel,
        out_type=jax.ShapeDtypeStruct((rows_per_chunk, EMBED), jnp.float32),
        mesh=mesh,
        scratch_types=[
            pltpu.VMEM((b_per_w,), jnp.int32),
            pltpu.VMEM((_NB, _CH, EMBED), jnp.float32),
            [pltpu.SemaphoreType.DMA] * _NB,
            [pltpu.SemaphoreType.DMA] * _NB,
        ],
    )
    def k(table_hbm, idx_hbm, out_hbm, idx_v, bufs, gsems, wsems):
        wid = lax.axis_index("s") * 2 + lax.axis_index("c")
        gwid = chunk * _NUM_WORKERS + wid
        brow = gwid // rows_per_batch
        bcol = (gwid % rows_per_batch) * b_per_w
        pltpu.sync_copy(idx_hbm.at[brow, pl.ds(bcol, b_per_w)], idx_v)

        def gather(c, s):
            return pltpu.async_copy(
                table_hbm.at[idx_v.at[pl.ds(c * _CH, _CH)]], bufs.at[s], gsems[s]
            )

        gs = [None] * n_ch
        ws = [None] * n_ch
        for s in range(min(_NB, n_ch)):
            gs[s] = gather(s, s)
        for c in range(n_ch):
            s = c % _NB
            gs[c].wait()
            ws[c] = pltpu.async_copy(
                bufs.at[s], out_hbm.at[pl.ds(wid * b_per_w + c * _CH, _CH)], wsems[s]
            )
            if c + _NB < n_ch:
                ws[c].wait()
                gs[c + _NB] = gather(c + _NB, s)
        for c in range(max(0, n_ch - _NB), n_ch):
            ws[c].wait()

    return k(table, ids2d)


def _tc_project_chunk(x, positions, w, b, buf, chunk, n_total):
    rows = x.shape[0]
    tiles = rows // _TB
    base_tile = chunk * tiles

    def body(buf_ref, x_ref, pos_ref, w_ref, b_ref, o_ref):
        del buf_ref
        xx = (x_ref[...] + pos_ref[1:2, :]).astype(jnp.bfloat16)
        w16 = w_ref[...].astype(jnp.bfloat16)
        o_ref[...] = (
            jnp.dot(xx, w16, preferred_element_type=jnp.float32) + b_ref[...]
        )

    if buf is None:
        buf = jnp.zeros((8, HIDDEN), jnp.float32)
        aliases = {}
    else:
        aliases = {0: 0}

    return pl.pallas_call(
        body,
        grid=(tiles,),
        in_specs=[
            pl.BlockSpec(memory_space=pl.ANY),
            pl.BlockSpec((_TB, EMBED), lambda i: (i, 0)),
            pl.BlockSpec((8, EMBED), lambda i: (0, 0)),
            pl.BlockSpec((EMBED, HIDDEN), lambda i: (0, 0)),
            pl.BlockSpec((1, HIDDEN), lambda i: (0, 0)),
        ],
        out_specs=pl.BlockSpec((_TB, HIDDEN), lambda i: (base_tile + i, 0)),
        out_shape=jax.ShapeDtypeStruct((n_total, HIDDEN), jnp.float32),
        input_output_aliases=aliases,
    )(buf, x, positions, w, b)


def kernel(input_ids, embed_tokens_w, embed_positions_w, proj_w, proj_b):
    batch, seq = input_ids.shape
    b_total = batch * seq
    rows_per_chunk = b_total // _N_CHUNKS
    ids = input_ids.astype(jnp.int32)
    b2d = proj_b.reshape(1, HIDDEN)

    gathered = [
        _sc_gather_chunk(embed_tokens_w, ids, c, rows_per_chunk)
        for c in range(_N_CHUNKS)
    ]
    buf = None
    for c in range(_N_CHUNKS):
        buf = _tc_project_chunk(
            gathered[c], embed_positions_w, proj_w, b2d, buf, c, b_total
        )
    return buf.reshape(batch, seq, HIDDEN)
